# baseline (speedup 1.0000x reference)
import functools

import jax
import jax.numpy as jnp
from jax import lax
from jax.experimental import pallas as pl
from jax.experimental.pallas import tpu as pltpu
from jax.experimental.pallas import tpu_sc as plsc

VOCAB = 100000
DIM = 128
HID = 1024
N_DENO = 41
N_CONO = 2
B = 16384
L = 50

NC = 2
NS = 16
NW = NC * NS

BL = B * L
ROWS_PER_W = BL // NW
B_PER_W = B // NW
BW_B = 128
LSPAN = 2
BGROUPS = B_PER_W // BW_B
LGROUPS = L // LSPAN
CHUNKS = BGROUPS * LGROUPS

BT = 1024
G = B // BT
NEG = -1e30
LOG2 = 0.6931471805599453


def _sc_body(table_hbm, idx_hbm, vecs_hbm, repr_hbm,
             idx_v, rows_a, rows_b, acc_v, gsa, gsb, wsa, wsb):
    wid = lax.axis_index("c") * NS + lax.axis_index("s")
    bbase = wid * B_PER_W
    pltpu.sync_copy(idx_hbm.at[wid], idx_v)

    def zero_acc():
        def zbody(j, carry):
            for d in range(8):
                acc_v[j, pl.ds(d * 16, 16)] = jnp.zeros((16,), jnp.float32)
            return carry
        lax.fori_loop(0, BW_B, zbody, 0)

    def gather_fire(c, rows, sem):
        for s in range(LSPAN):
            pltpu.async_copy(table_hbm.at[idx_v.at[c, s]],
                             rows.at[pl.ds(s * BW_B, BW_B)], sem)

    def gather_wait(rows, sem):
        for s in range(LSPAN):
            pltpu.make_async_copy(table_hbm.at[idx_v.at[0, 0]],
                                  rows.at[pl.ds(s * BW_B, BW_B)], sem).wait()

    def write_fire(c, rows, sem):
        h = c % LGROUPS
        g = c // LGROUPS
        for s in range(LSPAN):
            pltpu.async_copy(
                rows.at[pl.ds(s * BW_B, BW_B)],
                vecs_hbm.at[h * LSPAN + s, pl.ds(bbase + g * BW_B, BW_B)], sem)

    def write_wait(rows, sem):
        for s in range(LSPAN):
            pltpu.make_async_copy(rows.at[pl.ds(s * BW_B, BW_B)],
                                  vecs_hbm.at[0, pl.ds(0, BW_B)], sem).wait()

    def process(c, rows):
        @plsc.parallel_loop(0, BW_B, unroll=8)
        def jbody(j):
            for d in range(8):
                v = rows[j, pl.ds(d * 16, 16)]
                for s in range(1, LSPAN):
                    v = v + rows[s * BW_B + j, pl.ds(d * 16, 16)]
                plsc.addupdate(acc_v.at[j, pl.ds(d * 16, 16)], v)

        @pl.when(c % LGROUPS == LGROUPS - 1)
        def _flush():
            g = c // LGROUPS
            pltpu.sync_copy(acc_v,
                            repr_hbm.at[pl.ds(bbase + g * BW_B, BW_B)])
            zero_acc()

    zero_acc()
    gather_fire(0, rows_a, gsa)
    gather_fire(1, rows_b, gsb)

    def body(j, carry):
        ca = 2 * j
        cb = 2 * j + 1
        gather_wait(rows_a, gsa)
        process(ca, rows_a)
        write_fire(ca, rows_a, wsa)
        gather_wait(rows_b, gsb)
        process(cb, rows_b)
        write_fire(cb, rows_b, wsb)
        write_wait(rows_a, wsa)
        gather_fire(ca + 2, rows_a, gsa)
        write_wait(rows_b, wsb)
        gather_fire(cb + 2, rows_b, gsb)
        return carry

    lax.fori_loop(0, CHUNKS // 2 - 1, body, 0)
    gather_wait(rows_a, gsa)
    process(CHUNKS - 2, rows_a)
    write_fire(CHUNKS - 2, rows_a, wsa)
    gather_wait(rows_b, gsb)
    process(CHUNKS - 1, rows_b)
    write_fire(CHUNKS - 1, rows_b, wsb)
    write_wait(rows_a, wsa)
    write_wait(rows_b, wsb)


_sc_gather = functools.partial(
    pl.kernel,
    mesh=plsc.VectorSubcoreMesh(core_axis_name="c", subcore_axis_name="s"),
    out_type=[
        jax.ShapeDtypeStruct((L, B, DIM), jnp.float32),
        jax.ShapeDtypeStruct((B, DIM), jnp.float32),
    ],
    scratch_types=[
        pltpu.VMEM((CHUNKS, LSPAN, BW_B), jnp.int32),
        pltpu.VMEM((LSPAN * BW_B, DIM), jnp.float32),
        pltpu.VMEM((LSPAN * BW_B, DIM), jnp.float32),
        pltpu.VMEM((BW_B, DIM), jnp.float32),
        pltpu.SemaphoreType.DMA,
        pltpu.SemaphoreType.DMA,
        pltpu.SemaphoreType.DMA,
        pltpu.SemaphoreType.DMA,
    ],
)(_sc_body)


def _tc_body(x_ref, dW1_ref, db1_ref, dW2_ref, db2_ref, dW3_ref, db3_ref,
             cW1_ref, cb1_ref, cW2_ref, cb2_ref, cW3_ref, cb3_ref,
             dlab_ref, clab_ref, dsum_ref, csum_ref, asum_ref):
    i = pl.program_id(0)

    @pl.when(i == 0)
    def _init():
        dsum_ref[...] = jnp.zeros_like(dsum_ref)
        csum_ref[...] = jnp.zeros_like(csum_ref)
        asum_ref[...] = jnp.zeros_like(asum_ref)

    x = x_ref[...] * (1.0 / L)

    def probe(W1, b1, W2, b2, W3, b3):
        h = jnp.maximum(
            jnp.dot(x, W1[...], preferred_element_type=jnp.float32) + b1[...], 0.0)
        h = jnp.maximum(
            jnp.dot(h, W2[...], preferred_element_type=jnp.float32) + b2[...], 0.0)
        return jnp.dot(h, W3[...], preferred_element_type=jnp.float32) + b3[...]

    dlogits = probe(dW1_ref, db1_ref, dW2_ref, db2_ref, dW3_ref, db3_ref)
    clogits = probe(cW1_ref, cb1_ref, cW2_ref, cb2_ref, cW3_ref, cb3_ref)

    col = lax.broadcasted_iota(jnp.int32, (BT, 128), 1)

    def lse(lg):
        m = jnp.max(lg, axis=1, keepdims=True)
        return jnp.log(jnp.sum(jnp.exp(lg - m), axis=1, keepdims=True)) + m

    dlse = lse(dlogits)
    clse = lse(clogits)
    dpick = jnp.sum(jnp.where(col == dlab_ref[0], dlogits, 0.0), axis=1,
                    keepdims=True)
    cpick = jnp.sum(jnp.where(col == clab_ref[0], clogits, 0.0), axis=1,
                    keepdims=True)
    c2 = jnp.sum(jnp.where(col < N_CONO, clogits, 0.0), axis=1, keepdims=True)

    dsum_ref[...] += jnp.sum(dlse - dpick).reshape(1, 1)
    csum_ref[...] += jnp.sum(clse - cpick).reshape(1, 1)
    asum_ref[...] += jnp.sum((clse - 0.5 * c2) - LOG2).reshape(1, 1)


def _tc_losses(seq_repr, dW1, db1, dW2, db2, dW3p, db3p,
               cW1, cb1, cW2, cb2, cW3p, cb3p, dlab3, clab3):
    full = lambda shape: pl.BlockSpec(shape, lambda i: tuple(0 for _ in shape))
    return pl.pallas_call(
        _tc_body,
        grid=(G,),
        in_specs=[
            pl.BlockSpec((BT, DIM), lambda i: (i, 0)),
            full((DIM, HID)), full((1, HID)),
            full((HID, HID)), full((1, HID)),
            full((HID, 128)), full((1, 128)),
            full((DIM, HID)), full((1, HID)),
            full((HID, HID)), full((1, HID)),
            full((HID, 128)), full((1, 128)),
            pl.BlockSpec((1, BT, 1), lambda i: (i, 0, 0)),
            pl.BlockSpec((1, BT, 1), lambda i: (i, 0, 0)),
        ],
        out_specs=[pl.BlockSpec((1, 1), lambda i: (0, 0))] * 3,
        out_shape=[jax.ShapeDtypeStruct((1, 1), jnp.float32)] * 3,
    )(seq_repr, dW1, db1, dW2, db2, dW3p, db3p,
      cW1, cb1, cW2, cb2, cW3p, cb3p, dlab3, clab3)


def kernel(table, dW1, db1, dW2, db2, dW3, db3, cW1, cb1, cW2, cb2, cW3, cb3,
           seq_word_ids, deno_labels, cono_labels):
    idx_t = seq_word_ids.astype(jnp.int32).T
    idx5 = (idx_t.reshape(LGROUPS, LSPAN, NW, BGROUPS, BW_B)
            .transpose(2, 3, 0, 1, 4)
            .reshape(NW, CHUNKS, LSPAN, BW_B))

    vecs_t, seq_repr = _sc_gather(table, idx5)
    vecs = jnp.transpose(vecs_t, (1, 0, 2))

    dW3p = jnp.pad(dW3, ((0, 0), (0, 128 - N_DENO)))
    db3p = jnp.pad(db3, (0, 128 - N_DENO), constant_values=NEG).reshape(1, 128)
    cW3p = jnp.pad(cW3, ((0, 0), (0, 128 - N_CONO)))
    cb3p = jnp.pad(cb3, (0, 128 - N_CONO), constant_values=NEG).reshape(1, 128)
    dlab3 = deno_labels.reshape(G, BT, 1)
    clab3 = cono_labels.reshape(G, BT, 1)

    dsum, csum, asum = _tc_losses(
        seq_repr, dW1, db1.reshape(1, HID), dW2, db2.reshape(1, HID),
        dW3p, db3p, cW1, cb1.reshape(1, HID), cW2, cb2.reshape(1, HID),
        cW3p, cb3p, dlab3, clab3)

    deno_probe_loss = dsum[0, 0] / B
    cono_probe_loss = csum[0, 0] / B
    cono_adversary_loss = asum[0, 0] / B

    return (deno_probe_loss, cono_probe_loss, cono_adversary_loss, vecs)


# speedup vs baseline: 1.0179x; 1.0179x over previous
import functools

import jax
import jax.numpy as jnp
from jax import lax
from jax.experimental import pallas as pl
from jax.experimental.pallas import tpu as pltpu
from jax.experimental.pallas import tpu_sc as plsc

VOCAB = 100000
DIM = 128
HID = 1024
N_DENO = 41
N_CONO = 2
B = 16384
L = 50

NC = 2
NS = 16
NW = NC * NS

BL = B * L
ROWS_PER_W = BL // NW
B_PER_W = B // NW
BW_B = 128
LSPAN = 2
BGROUPS = B_PER_W // BW_B
LGROUPS = L // LSPAN
CHUNKS = BGROUPS * LGROUPS

BT = 2048
G = B // BT
NEG = -1e30
LOG2 = 0.6931471805599453


def _sc_body(table_hbm, idx_hbm, vecs_hbm, repr_hbm,
             idx_v, rows_a, rows_b, acc_v, gsa, gsb, wsa, wsb):
    wid = lax.axis_index("c") * NS + lax.axis_index("s")
    bbase = wid * B_PER_W
    pltpu.sync_copy(idx_hbm.at[wid], idx_v)

    def zero_acc():
        def zbody(j, carry):
            for d in range(8):
                acc_v[j, pl.ds(d * 16, 16)] = jnp.zeros((16,), jnp.float32)
            return carry
        lax.fori_loop(0, BW_B, zbody, 0)

    def gather_fire(c, rows, sem):
        for s in range(LSPAN):
            pltpu.async_copy(table_hbm.at[idx_v.at[c, s]],
                             rows.at[pl.ds(s * BW_B, BW_B)], sem)

    def gather_wait(rows, sem):
        for s in range(LSPAN):
            pltpu.make_async_copy(table_hbm.at[idx_v.at[0, 0]],
                                  rows.at[pl.ds(s * BW_B, BW_B)], sem).wait()

    def write_fire(c, rows, sem):
        h = c % LGROUPS
        g = c // LGROUPS
        for s in range(LSPAN):
            pltpu.async_copy(
                rows.at[pl.ds(s * BW_B, BW_B)],
                vecs_hbm.at[h * LSPAN + s, pl.ds(bbase + g * BW_B, BW_B)], sem)

    def write_wait(rows, sem):
        for s in range(LSPAN):
            pltpu.make_async_copy(rows.at[pl.ds(s * BW_B, BW_B)],
                                  vecs_hbm.at[0, pl.ds(0, BW_B)], sem).wait()

    def process(c, rows):
        @plsc.parallel_loop(0, BW_B, unroll=4)
        def jbody(j):
            for d in range(8):
                v = rows[j, pl.ds(d * 16, 16)]
                for s in range(1, LSPAN):
                    v = v + rows[s * BW_B + j, pl.ds(d * 16, 16)]
                plsc.addupdate(acc_v.at[j, pl.ds(d * 16, 16)], v)

        @pl.when(c % LGROUPS == LGROUPS - 1)
        def _flush():
            g = c // LGROUPS
            pltpu.sync_copy(acc_v,
                            repr_hbm.at[pl.ds(bbase + g * BW_B, BW_B)])
            zero_acc()

    zero_acc()
    gather_fire(0, rows_a, gsa)
    gather_fire(1, rows_b, gsb)

    def body(j, carry):
        ca = 2 * j
        cb = 2 * j + 1
        gather_wait(rows_a, gsa)
        process(ca, rows_a)
        write_fire(ca, rows_a, wsa)
        gather_wait(rows_b, gsb)
        process(cb, rows_b)
        write_fire(cb, rows_b, wsb)
        write_wait(rows_a, wsa)
        gather_fire(ca + 2, rows_a, gsa)
        write_wait(rows_b, wsb)
        gather_fire(cb + 2, rows_b, gsb)
        return carry

    lax.fori_loop(0, CHUNKS // 2 - 1, body, 0)
    gather_wait(rows_a, gsa)
    process(CHUNKS - 2, rows_a)
    write_fire(CHUNKS - 2, rows_a, wsa)
    gather_wait(rows_b, gsb)
    process(CHUNKS - 1, rows_b)
    write_fire(CHUNKS - 1, rows_b, wsb)
    write_wait(rows_a, wsa)
    write_wait(rows_b, wsb)


_sc_gather = functools.partial(
    pl.kernel,
    mesh=plsc.VectorSubcoreMesh(core_axis_name="c", subcore_axis_name="s"),
    out_type=[
        jax.ShapeDtypeStruct((L, B, DIM), jnp.float32),
        jax.ShapeDtypeStruct((B, DIM), jnp.float32),
    ],
    scratch_types=[
        pltpu.VMEM((CHUNKS, LSPAN, BW_B), jnp.int32),
        pltpu.VMEM((LSPAN * BW_B, DIM), jnp.float32),
        pltpu.VMEM((LSPAN * BW_B, DIM), jnp.float32),
        pltpu.VMEM((BW_B, DIM), jnp.float32),
        pltpu.SemaphoreType.DMA,
        pltpu.SemaphoreType.DMA,
        pltpu.SemaphoreType.DMA,
        pltpu.SemaphoreType.DMA,
    ],
)(_sc_body)


def _tc_body(x_ref, dW1_ref, db1_ref, dW2_ref, db2_ref, dW3_ref, db3_ref,
             cW1_ref, cb1_ref, cW2_ref, cb2_ref, cW3_ref, cb3_ref,
             dlab_ref, clab_ref, dsum_ref, csum_ref, asum_ref):
    i = pl.program_id(0)

    @pl.when(i == 0)
    def _init():
        dsum_ref[...] = jnp.zeros_like(dsum_ref)
        csum_ref[...] = jnp.zeros_like(csum_ref)
        asum_ref[...] = jnp.zeros_like(asum_ref)

    x = x_ref[...] * (1.0 / L)

    def probe(W1, b1, W2, b2, W3, b3):
        h = jnp.maximum(
            jnp.dot(x, W1[...], preferred_element_type=jnp.float32) + b1[...], 0.0)
        h = jnp.maximum(
            jnp.dot(h, W2[...], preferred_element_type=jnp.float32) + b2[...], 0.0)
        return jnp.dot(h, W3[...], preferred_element_type=jnp.float32) + b3[...]

    dlogits = probe(dW1_ref, db1_ref, dW2_ref, db2_ref, dW3_ref, db3_ref)
    clogits = probe(cW1_ref, cb1_ref, cW2_ref, cb2_ref, cW3_ref, cb3_ref)

    col = lax.broadcasted_iota(jnp.int32, (BT, 128), 1)

    def lse(lg):
        m = jnp.max(lg, axis=1, keepdims=True)
        return jnp.log(jnp.sum(jnp.exp(lg - m), axis=1, keepdims=True)) + m

    dlse = lse(dlogits)
    clse = lse(clogits)
    dpick = jnp.sum(jnp.where(col == dlab_ref[0], dlogits, 0.0), axis=1,
                    keepdims=True)
    cpick = jnp.sum(jnp.where(col == clab_ref[0], clogits, 0.0), axis=1,
                    keepdims=True)
    c2 = jnp.sum(jnp.where(col < N_CONO, clogits, 0.0), axis=1, keepdims=True)

    dsum_ref[...] += jnp.sum(dlse - dpick).reshape(1, 1)
    csum_ref[...] += jnp.sum(clse - cpick).reshape(1, 1)
    asum_ref[...] += jnp.sum((clse - 0.5 * c2) - LOG2).reshape(1, 1)


def _tc_losses(seq_repr, dW1, db1, dW2, db2, dW3p, db3p,
               cW1, cb1, cW2, cb2, cW3p, cb3p, dlab3, clab3):
    full = lambda shape: pl.BlockSpec(shape, lambda i: tuple(0 for _ in shape))
    return pl.pallas_call(
        _tc_body,
        grid=(G,),
        in_specs=[
            pl.BlockSpec((BT, DIM), lambda i: (i, 0)),
            full((DIM, HID)), full((1, HID)),
            full((HID, HID)), full((1, HID)),
            full((HID, 128)), full((1, 128)),
            full((DIM, HID)), full((1, HID)),
            full((HID, HID)), full((1, HID)),
            full((HID, 128)), full((1, 128)),
            pl.BlockSpec((1, BT, 1), lambda i: (i, 0, 0)),
            pl.BlockSpec((1, BT, 1), lambda i: (i, 0, 0)),
        ],
        out_specs=[pl.BlockSpec((1, 1), lambda i: (0, 0))] * 3,
        out_shape=[jax.ShapeDtypeStruct((1, 1), jnp.float32)] * 3,
    )(seq_repr, dW1, db1, dW2, db2, dW3p, db3p,
      cW1, cb1, cW2, cb2, cW3p, cb3p, dlab3, clab3)


def kernel(table, dW1, db1, dW2, db2, dW3, db3, cW1, cb1, cW2, cb2, cW3, cb3,
           seq_word_ids, deno_labels, cono_labels):
    idx_t = seq_word_ids.astype(jnp.int32).T
    idx5 = (idx_t.reshape(LGROUPS, LSPAN, NW, BGROUPS, BW_B)
            .transpose(2, 3, 0, 1, 4)
            .reshape(NW, CHUNKS, LSPAN, BW_B))

    vecs_t, seq_repr = _sc_gather(table, idx5)
    vecs = jnp.transpose(vecs_t, (1, 0, 2))

    dW3p = jnp.pad(dW3, ((0, 0), (0, 128 - N_DENO)))
    db3p = jnp.pad(db3, (0, 128 - N_DENO), constant_values=NEG).reshape(1, 128)
    cW3p = jnp.pad(cW3, ((0, 0), (0, 128 - N_CONO)))
    cb3p = jnp.pad(cb3, (0, 128 - N_CONO), constant_values=NEG).reshape(1, 128)
    dlab3 = deno_labels.reshape(G, BT, 1)
    clab3 = cono_labels.reshape(G, BT, 1)

    dsum, csum, asum = _tc_losses(
        seq_repr, dW1, db1.reshape(1, HID), dW2, db2.reshape(1, HID),
        dW3p, db3p, cW1, cb1.reshape(1, HID), cW2, cb2.reshape(1, HID),
        cW3p, cb3p, dlab3, clab3)

    deno_probe_loss = dsum[0, 0] / B
    cono_probe_loss = csum[0, 0] / B
    cono_adversary_loss = asum[0, 0] / B

    return (deno_probe_loss, cono_probe_loss, cono_adversary_loss, vecs)
